# baseline (speedup 1.0000x reference)
import functools

import jax
import jax.numpy as jnp
from jax import lax
from jax.experimental import pallas as pl
from jax.experimental.pallas import tpu as pltpu
from jax.experimental.pallas import tpu_sc as plsc

N_NODES = 10000
DIM = 128
N_REL = 4
N_BLK = 4
BS = 32
N_EDGES = 160000

NW = 32
CHUNK = 128
E2 = 2 * N_EDGES
K = 4 * (-(-E2 // (NW * CHUNK * 4)))
E2P = NW * K * CHUNK
PAD = E2P - E2
ACC_ROWS = 10112
ROWS_PER_TILE = ACC_ROWS // 16


def _transform_body(x_ref, m_ref, blk_ref, y_ref, s_ref):
    xb = x_ref[...]
    for r in range(N_REL + 1):
        parts = []
        for b in range(N_BLK):
            parts.append(
                jnp.dot(xb[:, b * BS:(b + 1) * BS], blk_ref[r, b],
                        preferred_element_type=jnp.float32))
        yr = jnp.concatenate(parts, axis=1)
        if r < N_REL:
            y_ref[r] = yr
        else:
            s_ref[...] = yr * m_ref[...]


def _transform(x, maskf, blocks):
    br = ACC_ROWS // 8
    return pl.pallas_call(
        _transform_body,
        grid=(8,),
        in_specs=[
            pl.BlockSpec((br, DIM), lambda i: (i, 0)),
            pl.BlockSpec((br, 1), lambda i: (i, 0)),
            pl.BlockSpec((N_REL + 1, N_BLK, BS, BS), lambda i: (0, 0, 0, 0)),
        ],
        out_specs=[
            pl.BlockSpec((N_REL, br, DIM), lambda i: (0, i, 0)),
            pl.BlockSpec((br, DIM), lambda i: (i, 0)),
        ],
        out_shape=[
            jax.ShapeDtypeStruct((N_REL, N_NODES, DIM), jnp.float32),
            jax.ShapeDtypeStruct((ACC_ROWS, DIM), jnp.float32),
        ],
    )(x, maskf, blocks)


def _sc_scatter(yflat, idx, spad, zeros):
    mesh = plsc.VectorSubcoreMesh(core_axis_name="c", subcore_axis_name="s")

    @functools.partial(
        pl.kernel,
        mesh=mesh,
        out_type=jax.ShapeDtypeStruct((2, ACC_ROWS, DIM), jnp.float32),
        scratch_types=[
            pltpu.VMEM((4, 2, CHUNK), jnp.int32),
            pltpu.VMEM((CHUNK, DIM), jnp.float32),
            pltpu.VMEM((CHUNK, DIM), jnp.float32),
            pltpu.VMEM_SHARED((ACC_ROWS, DIM), jnp.float32),
            pltpu.SemaphoreType.DMA,
            pltpu.SemaphoreType.DMA,
            pltpu.SemaphoreType.DMA,
            pltpu.SemaphoreType.DMA,
            pltpu.SemaphoreType.DMA,
            pltpu.SemaphoreType.DMA,
            pltpu.SemaphoreType.DMA,
            pltpu.SemaphoreType.DMA,
        ],
    )
    def body(yflat_hbm, idx_hbm, spad_hbm, zeros_hbm, out_hbm,
             ring, rows_a, rows_b, acc, g0, g1, s0, s1, i0, i1, i2, i3):
        cid = lax.axis_index("c")
        sid = lax.axis_index("s")
        wid = cid * 16 + sid
        iw = idx_hbm.at[wid]
        rows = (rows_a, rows_b)
        gsem = (g0, g1)
        ssem = (s0, s1)
        isem = (i0, i1, i2, i3)

        for u in range(4):
            pltpu.async_copy(iw.at[u], ring.at[u], isem[u])
        pltpu.make_async_copy(iw.at[0], ring.at[0], isem[0]).wait()
        pltpu.async_copy(yflat_hbm.at[ring.at[0, 0]], rows[0], gsem[0])

        sl_init = pl.ds(sid * ROWS_PER_TILE, ROWS_PER_TILE)

        @pl.when(cid == 0)
        def _():
            pltpu.sync_copy(spad_hbm.at[sl_init], acc.at[sl_init])

        @pl.when(cid == 1)
        def _():
            pltpu.sync_copy(zeros_hbm.at[sl_init], acc.at[sl_init])

        plsc.subcore_barrier()


        def quad(i, carry):
            for u in range(4):
                j = 4 * i + u
                sl, slp, sl1 = u, (u + 3) % 4, (u + 1) % 4
                bj, bj1 = rows[u % 2], rows[(u + 1) % 2]
                pltpu.make_async_copy(yflat_hbm.at[ring.at[sl, 0]],
                                      bj, gsem[u % 2]).wait()
                pltpu.async_copy(bj, acc.at[ring.at[sl, 1]], ssem[u % 2],
                                 add=True)

                @pl.when(j > 0)
                def _():
                    pltpu.make_async_copy(bj1, acc.at[ring.at[slp, 1]],
                                          ssem[(u + 1) % 2]).wait()

                @pl.when(jnp.logical_and(j > 0, j + 3 < K))
                def _():
                    pltpu.async_copy(iw.at[j + 3], ring.at[slp], isem[slp])

                @pl.when(j + 1 < K)
                def _():
                    pltpu.make_async_copy(iw.at[j + 1], ring.at[sl1],
                                          isem[sl1]).wait()
                    pltpu.async_copy(yflat_hbm.at[ring.at[sl1, 0]],
                                     bj1, gsem[(u + 1) % 2])
            return carry

        lax.fori_loop(0, K // 4, quad, 0)
        pltpu.make_async_copy(rows[(K - 1) % 2],
                              acc.at[ring.at[(K - 1) % 4, 1]],
                              ssem[(K - 1) % 2]).wait()
        plsc.subcore_barrier()
        pltpu.sync_copy(
            acc.at[pl.ds(sid * ROWS_PER_TILE, ROWS_PER_TILE)],
            out_hbm.at[cid].at[pl.ds(sid * ROWS_PER_TILE, ROWS_PER_TILE)])

    return body(yflat, idx, spad, zeros)


def _combine_body(p0_ref, p1_ref, o_ref):
    o_ref[...] = p0_ref[...] + p1_ref[...]


def _combine(p0, p1):
    br = 2000
    spec = pl.BlockSpec((br, DIM), lambda i: (i, 0))
    return pl.pallas_call(
        _combine_body,
        grid=(N_NODES // br,),
        in_specs=[spec, spec],
        out_specs=spec,
        out_shape=jax.ShapeDtypeStruct((N_NODES, DIM), jnp.float32),
    )(p0, p1)


def kernel(x, node_keep_mask, source, target, edge_type, blocks):
    maskf = node_keep_mask.astype(jnp.float32).reshape(N_NODES, 1)
    source = source.astype(jnp.int32)
    target = target.astype(jnp.int32)
    edge_type = edge_type.astype(jnp.int32)

    gidx = jnp.concatenate([edge_type * N_NODES + source,
                            edge_type * N_NODES + target])
    sidx = jnp.concatenate([target, source])
    pad = jnp.arange(PAD, dtype=jnp.int32)
    gidx = jnp.concatenate([gidx, (pad * 131) % (N_REL * N_NODES)])
    sidx = jnp.concatenate([sidx, N_NODES + (pad % (ACC_ROWS - N_NODES))])
    idx = jnp.stack([gidx.reshape(NW, K, CHUNK),
                     sidx.reshape(NW, K, CHUNK)], axis=2)

    y, spad = _transform(x, maskf, blocks)
    yflat = y.reshape(N_REL * N_NODES, DIM)
    zeros = jnp.zeros((ACC_ROWS, DIM), jnp.float32)
    p = _sc_scatter(yflat, idx, spad, zeros)
    return _combine(p[0], p[1])


# speedup vs baseline: 1.0095x; 1.0095x over previous
import functools

import jax
import jax.numpy as jnp
from jax import lax
from jax.experimental import pallas as pl
from jax.experimental.pallas import tpu as pltpu
from jax.experimental.pallas import tpu_sc as plsc

N_NODES = 10000
DIM = 128
N_REL = 4
N_BLK = 4
BS = 32
N_EDGES = 160000

NW = 32
CHUNK = 128
E2 = 2 * N_EDGES
K = 4 * (-(-E2 // (NW * CHUNK * 4)))
E2P = NW * K * CHUNK
PAD = E2P - E2
ACC_ROWS = 10112
ROWS_PER_TILE = ACC_ROWS // 16


def _transform_body(x_ref, m_ref, blk_ref, y_ref):
    r = pl.program_id(1)
    xb = x_ref[...]
    z = jnp.zeros((BS, BS), jnp.float32)
    w = jnp.concatenate([
        jnp.concatenate([blk_ref[0, b] if j == b else z
                         for j in range(N_BLK)], axis=1)
        for b in range(N_BLK)], axis=0)
    yr = jnp.dot(xb, w, preferred_element_type=jnp.float32)
    y_ref[...] = jnp.where(r == N_REL, yr * m_ref[...], yr)


def _transform(x, maskf, blocks):
    br = ACC_ROWS // 8
    return pl.pallas_call(
        _transform_body,
        grid=(8, N_REL + 1),
        in_specs=[
            pl.BlockSpec((br, DIM), lambda i, r: (i, 0)),
            pl.BlockSpec((br, 1), lambda i, r: (i, 0)),
            pl.BlockSpec((1, N_BLK, BS, BS), lambda i, r: (r, 0, 0, 0)),
        ],
        out_specs=pl.BlockSpec((br, DIM), lambda i, r: (r * 8 + i, 0)),
        out_shape=jax.ShapeDtypeStruct(((N_REL + 1) * ACC_ROWS, DIM),
                                       jnp.float32),
    )(x, maskf, blocks)


SOFF = N_REL * ACC_ROWS


def _sc_scatter(yflat, idx, zeros):
    mesh = plsc.VectorSubcoreMesh(core_axis_name="c", subcore_axis_name="s")

    @functools.partial(
        pl.kernel,
        mesh=mesh,
        out_type=jax.ShapeDtypeStruct((2, ACC_ROWS, DIM), jnp.float32),
        scratch_types=[
            pltpu.VMEM((4, 2, CHUNK), jnp.int32),
            pltpu.VMEM((CHUNK, DIM), jnp.float32),
            pltpu.VMEM((CHUNK, DIM), jnp.float32),
            pltpu.VMEM_SHARED((ACC_ROWS, DIM), jnp.float32),
            pltpu.SemaphoreType.DMA,
            pltpu.SemaphoreType.DMA,
            pltpu.SemaphoreType.DMA,
            pltpu.SemaphoreType.DMA,
            pltpu.SemaphoreType.DMA,
            pltpu.SemaphoreType.DMA,
            pltpu.SemaphoreType.DMA,
            pltpu.SemaphoreType.DMA,
        ],
    )
    def body(yflat_hbm, idx_hbm, zeros_hbm, out_hbm,
             ring, rows_a, rows_b, acc, g0, g1, s0, s1, i0, i1, i2, i3):
        cid = lax.axis_index("c")
        sid = lax.axis_index("s")
        wid = cid * 16 + sid
        iw = idx_hbm.at[wid]
        rows = (rows_a, rows_b)
        gsem = (g0, g1)
        ssem = (s0, s1)
        isem = (i0, i1, i2, i3)

        for u in range(4):
            pltpu.async_copy(iw.at[u], ring.at[u], isem[u])
        pltpu.make_async_copy(iw.at[0], ring.at[0], isem[0]).wait()
        pltpu.async_copy(yflat_hbm.at[ring.at[0, 0]], rows[0], gsem[0])

        sl_init = pl.ds(sid * ROWS_PER_TILE, ROWS_PER_TILE)

        @pl.when(cid == 0)
        def _():
            pltpu.sync_copy(
                yflat_hbm.at[pl.ds(SOFF + sid * ROWS_PER_TILE,
                                   ROWS_PER_TILE)],
                acc.at[sl_init])

        @pl.when(cid == 1)
        def _():
            pltpu.sync_copy(zeros_hbm.at[sl_init], acc.at[sl_init])

        plsc.subcore_barrier()


        def quad(i, carry):
            for u in range(4):
                j = 4 * i + u
                sl, slp, sl1 = u, (u + 3) % 4, (u + 1) % 4
                bj, bj1 = rows[u % 2], rows[(u + 1) % 2]
                pltpu.make_async_copy(yflat_hbm.at[ring.at[sl, 0]],
                                      bj, gsem[u % 2]).wait()
                pltpu.async_copy(bj, acc.at[ring.at[sl, 1]], ssem[u % 2],
                                 add=True)

                @pl.when(j > 0)
                def _():
                    pltpu.make_async_copy(bj1, acc.at[ring.at[slp, 1]],
                                          ssem[(u + 1) % 2]).wait()

                @pl.when(jnp.logical_and(j > 0, j + 3 < K))
                def _():
                    pltpu.async_copy(iw.at[j + 3], ring.at[slp], isem[slp])

                @pl.when(j + 1 < K)
                def _():
                    pltpu.make_async_copy(iw.at[j + 1], ring.at[sl1],
                                          isem[sl1]).wait()
                    pltpu.async_copy(yflat_hbm.at[ring.at[sl1, 0]],
                                     bj1, gsem[(u + 1) % 2])
            return carry

        lax.fori_loop(0, K // 4, quad, 0)
        pltpu.make_async_copy(rows[(K - 1) % 2],
                              acc.at[ring.at[(K - 1) % 4, 1]],
                              ssem[(K - 1) % 2]).wait()
        plsc.subcore_barrier()
        pltpu.sync_copy(
            acc.at[pl.ds(sid * ROWS_PER_TILE, ROWS_PER_TILE)],
            out_hbm.at[cid].at[pl.ds(sid * ROWS_PER_TILE, ROWS_PER_TILE)])

    return body(yflat, idx, zeros)


def _combine_body(p0_ref, p1_ref, o_ref):
    o_ref[...] = p0_ref[0] + p1_ref[0]


def _combine(p):
    br = 2000
    return pl.pallas_call(
        _combine_body,
        grid=(N_NODES // br,),
        in_specs=[
            pl.BlockSpec((1, br, DIM), lambda i: (0, i, 0)),
            pl.BlockSpec((1, br, DIM), lambda i: (1, i, 0)),
        ],
        out_specs=pl.BlockSpec((br, DIM), lambda i: (i, 0)),
        out_shape=jax.ShapeDtypeStruct((N_NODES, DIM), jnp.float32),
    )(p, p)


def kernel(x, node_keep_mask, source, target, edge_type, blocks):
    maskf = node_keep_mask.astype(jnp.float32).reshape(N_NODES, 1)
    source = source.astype(jnp.int32)
    target = target.astype(jnp.int32)
    edge_type = edge_type.astype(jnp.int32)

    gidx = jnp.concatenate([edge_type * ACC_ROWS + source,
                            edge_type * ACC_ROWS + target])
    sidx = jnp.concatenate([target, source])
    pad = jnp.arange(PAD, dtype=jnp.int32)
    gidx = jnp.concatenate([gidx, (pad * 131) % (N_REL * ACC_ROWS)])
    sidx = jnp.concatenate([sidx, N_NODES + (pad % (ACC_ROWS - N_NODES))])
    idx = jnp.stack([gidx.reshape(NW, K, CHUNK),
                     sidx.reshape(NW, K, CHUNK)], axis=2)

    yflat = _transform(x, maskf, blocks)
    zeros = jnp.zeros((ACC_ROWS, DIM), jnp.float32)
    p = _sc_scatter(yflat, idx, zeros)
    return _combine(p)


# speedup vs baseline: 1.1322x; 1.1215x over previous
import functools

import jax
import jax.numpy as jnp
from jax import lax
from jax.experimental import pallas as pl
from jax.experimental.pallas import tpu as pltpu
from jax.experimental.pallas import tpu_sc as plsc

N_NODES = 10000
DIM = 128
N_REL = 4
N_BLK = 4
BS = 32
N_EDGES = 160000

NW = 32
CHUNK = 128
E2 = 2 * N_EDGES
K = 4 * (-(-E2 // (NW * CHUNK * 4)))
E2P = NW * K * CHUNK
PAD = E2P - E2
ACC_ROWS = 10112
ROWS_PER_TILE = ACC_ROWS // 16


def _transform_body(x_ref, m_ref, blk_ref, y_ref):
    r = pl.program_id(0)
    xb = x_ref[...]
    z = jnp.zeros((BS, BS), jnp.float32)
    w = jnp.concatenate([
        jnp.concatenate([blk_ref[0, b] if j == b else z
                         for j in range(N_BLK)], axis=1)
        for b in range(N_BLK)], axis=0)
    yr = jnp.dot(xb, w, preferred_element_type=jnp.float32)
    y_ref[pl.ds(0, N_NODES), :] = jnp.where(r == N_REL, yr * m_ref[...], yr)


def _transform(x, maskf, blocks):
    return pl.pallas_call(
        _transform_body,
        grid=(N_REL + 1,),
        in_specs=[
            pl.BlockSpec((N_NODES, DIM), lambda r: (0, 0)),
            pl.BlockSpec((N_NODES, 1), lambda r: (0, 0)),
            pl.BlockSpec((1, N_BLK, BS, BS), lambda r: (r, 0, 0, 0)),
        ],
        out_specs=pl.BlockSpec((ACC_ROWS, DIM), lambda r: (r, 0)),
        out_shape=jax.ShapeDtypeStruct(((N_REL + 1) * ACC_ROWS, DIM),
                                       jnp.float32),
    )(x, maskf, blocks)


SOFF = N_REL * ACC_ROWS


def _sc_scatter(yflat, gidx, sidx, zeros):
    mesh = plsc.VectorSubcoreMesh(core_axis_name="c", subcore_axis_name="s")

    @functools.partial(
        pl.kernel,
        mesh=mesh,
        out_type=jax.ShapeDtypeStruct((2, ACC_ROWS, DIM), jnp.float32),
        scratch_types=[
            pltpu.VMEM((4, 2, CHUNK), jnp.int32),
            pltpu.VMEM((CHUNK, DIM), jnp.float32),
            pltpu.VMEM((CHUNK, DIM), jnp.float32),
            pltpu.VMEM_SHARED((ACC_ROWS, DIM), jnp.float32),
            pltpu.SemaphoreType.DMA,
            pltpu.SemaphoreType.DMA,
            pltpu.SemaphoreType.DMA,
            pltpu.SemaphoreType.DMA,
            pltpu.SemaphoreType.DMA,
            pltpu.SemaphoreType.DMA,
            pltpu.SemaphoreType.DMA,
            pltpu.SemaphoreType.DMA,
        ],
    )
    def body(yflat_hbm, gidx_hbm, sidx_hbm, zeros_hbm, out_hbm,
             ring, rows_a, rows_b, acc, g0, g1, s0, s1, i0, i1, i2, i3):
        cid = lax.axis_index("c")
        sid = lax.axis_index("s")
        wid = cid * 16 + sid
        gw = gidx_hbm.at[wid]
        sw = sidx_hbm.at[wid]
        rows = (rows_a, rows_b)
        gsem = (g0, g1)
        ssem = (s0, s1)
        isem = (i0, i1, i2, i3)

        def idx_start(row, slot):
            pltpu.async_copy(gw.at[row], ring.at[slot, 0], isem[slot])
            pltpu.async_copy(sw.at[row], ring.at[slot, 1], isem[slot])

        def idx_wait(row, slot):
            pltpu.make_async_copy(gw.at[row], ring.at[slot, 0],
                                  isem[slot]).wait()
            pltpu.make_async_copy(sw.at[row], ring.at[slot, 1],
                                  isem[slot]).wait()

        for u in range(4):
            idx_start(u, u)
        idx_wait(0, 0)
        pltpu.async_copy(yflat_hbm.at[ring.at[0, 0]], rows[0], gsem[0])

        sl_init = pl.ds(sid * ROWS_PER_TILE, ROWS_PER_TILE)

        @pl.when(cid == 0)
        def _():
            pltpu.sync_copy(
                yflat_hbm.at[pl.ds(SOFF + sid * ROWS_PER_TILE,
                                   ROWS_PER_TILE)],
                acc.at[sl_init])

        @pl.when(cid == 1)
        def _():
            pltpu.sync_copy(zeros_hbm.at[sl_init], acc.at[sl_init])

        plsc.subcore_barrier()


        def quad(i, carry):
            for u in range(4):
                j = 4 * i + u
                sl, slp, sl1 = u, (u + 3) % 4, (u + 1) % 4
                bj, bj1 = rows[u % 2], rows[(u + 1) % 2]
                pltpu.make_async_copy(yflat_hbm.at[ring.at[sl, 0]],
                                      bj, gsem[u % 2]).wait()
                pltpu.async_copy(bj, acc.at[ring.at[sl, 1]], ssem[u % 2],
                                 add=True)

                @pl.when(j > 0)
                def _():
                    pltpu.make_async_copy(bj1, acc.at[ring.at[slp, 1]],
                                          ssem[(u + 1) % 2]).wait()

                @pl.when(jnp.logical_and(j > 0, j + 3 < K))
                def _():
                    idx_start(j + 3, slp)

                @pl.when(j + 1 < K)
                def _():
                    idx_wait(j + 1, sl1)
                    pltpu.async_copy(yflat_hbm.at[ring.at[sl1, 0]],
                                     bj1, gsem[(u + 1) % 2])
            return carry

        lax.fori_loop(0, K // 4, quad, 0)
        pltpu.make_async_copy(rows[(K - 1) % 2],
                              acc.at[ring.at[(K - 1) % 4, 1]],
                              ssem[(K - 1) % 2]).wait()
        plsc.subcore_barrier()
        pltpu.sync_copy(
            acc.at[pl.ds(sid * ROWS_PER_TILE, ROWS_PER_TILE)],
            out_hbm.at[cid].at[pl.ds(sid * ROWS_PER_TILE, ROWS_PER_TILE)])

    return body(yflat, gidx, sidx, zeros)


def _combine_body(p0_ref, p1_ref, o_ref):
    o_ref[...] = p0_ref[0] + p1_ref[0]


def _combine(p):
    br = 2000
    return pl.pallas_call(
        _combine_body,
        grid=(N_NODES // br,),
        in_specs=[
            pl.BlockSpec((1, br, DIM), lambda i: (0, i, 0)),
            pl.BlockSpec((1, br, DIM), lambda i: (1, i, 0)),
        ],
        out_specs=pl.BlockSpec((br, DIM), lambda i: (i, 0)),
        out_shape=jax.ShapeDtypeStruct((N_NODES, DIM), jnp.float32),
    )(p, p)


def kernel(x, node_keep_mask, source, target, edge_type, blocks):
    maskf = node_keep_mask.astype(jnp.float32).reshape(N_NODES, 1)
    source = source.astype(jnp.int32)
    target = target.astype(jnp.int32)
    edge_type = edge_type.astype(jnp.int32)

    gidx = jnp.concatenate([edge_type * ACC_ROWS + source,
                            edge_type * ACC_ROWS + target])
    sidx = jnp.concatenate([target, source])
    pad = jnp.arange(PAD, dtype=jnp.int32)
    gidx = jnp.concatenate([gidx, (pad * 131) % (N_REL * ACC_ROWS)])
    sidx = jnp.concatenate([sidx, N_NODES + (pad % (ACC_ROWS - N_NODES))])

    yflat = _transform(x, maskf, blocks)
    zeros = jnp.zeros((ACC_ROWS, DIM), jnp.float32)
    p = _sc_scatter(yflat, gidx.reshape(NW, K, CHUNK),
                    sidx.reshape(NW, K, CHUNK), zeros)
    return _combine(p)


# speedup vs baseline: 1.1536x; 1.0189x over previous
import functools

import jax
import jax.numpy as jnp
from jax import lax
from jax.experimental import pallas as pl
from jax.experimental.pallas import tpu as pltpu
from jax.experimental.pallas import tpu_sc as plsc

N_NODES = 10000
DIM = 128
N_REL = 4
N_BLK = 4
BS = 32
N_EDGES = 160000

NW = 32
CHUNK = 128
E2 = 2 * N_EDGES
K = 4 * (-(-E2 // (NW * CHUNK * 4)))
E2P = NW * K * CHUNK
PAD = E2P - E2
ACC_ROWS = 10112
ROWS_PER_TILE = ACC_ROWS // 16


def _transform_body(x_ref, m_ref, blk_ref, y_ref):
    r = pl.program_id(0)
    xb = x_ref[...]
    z = jnp.zeros((BS, BS), jnp.float32)
    w = jnp.concatenate([
        jnp.concatenate([blk_ref[0, b] if j == b else z
                         for j in range(N_BLK)], axis=1)
        for b in range(N_BLK)], axis=0)
    yr = jnp.dot(xb, w, preferred_element_type=jnp.float32)
    y_ref[pl.ds(0, N_NODES), :] = jnp.where(r == N_REL, yr * m_ref[...], yr)


def _transform(x, maskf, blocks):
    return pl.pallas_call(
        _transform_body,
        grid=(N_REL + 1,),
        in_specs=[
            pl.BlockSpec((N_NODES, DIM), lambda r: (0, 0)),
            pl.BlockSpec((N_NODES, 1), lambda r: (0, 0)),
            pl.BlockSpec((1, N_BLK, BS, BS), lambda r: (r, 0, 0, 0)),
        ],
        out_specs=pl.BlockSpec((ACC_ROWS, DIM), lambda r: (r, 0)),
        out_shape=jax.ShapeDtypeStruct(((N_REL + 1) * ACC_ROWS, DIM),
                                       jnp.float32),
    )(x, maskf, blocks)


SOFF = N_REL * ACC_ROWS


def _sc_scatter(yflat, gidx, sidx, zeros):
    mesh = plsc.VectorSubcoreMesh(core_axis_name="c", subcore_axis_name="s")

    @functools.partial(
        pl.kernel,
        mesh=mesh,
        out_type=jax.ShapeDtypeStruct((2, ACC_ROWS, DIM), jnp.float32),
        scratch_types=[
            pltpu.VMEM((4, 2, CHUNK), jnp.int32),
            pltpu.VMEM((CHUNK, DIM), jnp.float32),
            pltpu.VMEM((CHUNK, DIM), jnp.float32),
            pltpu.VMEM_SHARED((ACC_ROWS, DIM), jnp.float32),
            pltpu.SemaphoreType.DMA,
            pltpu.SemaphoreType.DMA,
            pltpu.SemaphoreType.DMA,
            pltpu.SemaphoreType.DMA,
            pltpu.SemaphoreType.DMA,
            pltpu.SemaphoreType.DMA,
            pltpu.SemaphoreType.DMA,
            pltpu.SemaphoreType.DMA,
        ],
    )
    def body(yflat_hbm, gidx_hbm, sidx_hbm, zeros_hbm, out_hbm,
             ring, rows_a, rows_b, acc, g0, g1, s0, s1, i0, i1, i2, i3):
        cid = lax.axis_index("c")
        sid = lax.axis_index("s")
        wid = cid * 16 + sid
        base = wid * (K * CHUNK)
        rows = (rows_a, rows_b)
        gsem = (g0, g1)
        ssem = (s0, s1)
        isem = (i0, i1, i2, i3)

        def idx_start(row, slot):
            sl = pl.ds(base + row * CHUNK, CHUNK)
            pltpu.async_copy(gidx_hbm.at[sl], ring.at[slot, 0], isem[slot])
            pltpu.async_copy(sidx_hbm.at[sl], ring.at[slot, 1], isem[slot])

        def idx_wait(row, slot):
            sl = pl.ds(base + row * CHUNK, CHUNK)
            pltpu.make_async_copy(gidx_hbm.at[sl], ring.at[slot, 0],
                                  isem[slot]).wait()
            pltpu.make_async_copy(sidx_hbm.at[sl], ring.at[slot, 1],
                                  isem[slot]).wait()

        for u in range(4):
            idx_start(u, u)
        idx_wait(0, 0)
        pltpu.async_copy(yflat_hbm.at[ring.at[0, 0]], rows[0], gsem[0])

        sl_init = pl.ds(sid * ROWS_PER_TILE, ROWS_PER_TILE)

        @pl.when(cid == 0)
        def _():
            pltpu.sync_copy(
                yflat_hbm.at[pl.ds(SOFF + sid * ROWS_PER_TILE,
                                   ROWS_PER_TILE)],
                acc.at[sl_init])

        @pl.when(cid == 1)
        def _():
            pltpu.sync_copy(zeros_hbm.at[sl_init], acc.at[sl_init])

        plsc.subcore_barrier()


        def quad(i, carry):
            for u in range(4):
                j = 4 * i + u
                sl, slp, sl1 = u, (u + 3) % 4, (u + 1) % 4
                bj, bj1 = rows[u % 2], rows[(u + 1) % 2]
                pltpu.make_async_copy(yflat_hbm.at[ring.at[sl, 0]],
                                      bj, gsem[u % 2]).wait()
                pltpu.async_copy(bj, acc.at[ring.at[sl, 1]], ssem[u % 2],
                                 add=True)

                @pl.when(j > 0)
                def _():
                    pltpu.make_async_copy(bj1, acc.at[ring.at[slp, 1]],
                                          ssem[(u + 1) % 2]).wait()

                @pl.when(jnp.logical_and(j > 0, j + 3 < K))
                def _():
                    idx_start(j + 3, slp)

                @pl.when(j + 1 < K)
                def _():
                    idx_wait(j + 1, sl1)
                    pltpu.async_copy(yflat_hbm.at[ring.at[sl1, 0]],
                                     bj1, gsem[(u + 1) % 2])
            return carry

        lax.fori_loop(0, K // 4, quad, 0)
        pltpu.make_async_copy(rows[(K - 1) % 2],
                              acc.at[ring.at[(K - 1) % 4, 1]],
                              ssem[(K - 1) % 2]).wait()
        plsc.subcore_barrier()
        pltpu.sync_copy(
            acc.at[pl.ds(sid * ROWS_PER_TILE, ROWS_PER_TILE)],
            out_hbm.at[cid].at[pl.ds(sid * ROWS_PER_TILE, ROWS_PER_TILE)])

    return body(yflat, gidx, sidx, zeros)


def _combine_body(p0_ref, p1_ref, o_ref):
    o_ref[...] = p0_ref[0] + p1_ref[0]


def _combine(p):
    br = 2000
    return pl.pallas_call(
        _combine_body,
        grid=(N_NODES // br,),
        in_specs=[
            pl.BlockSpec((1, br, DIM), lambda i: (0, i, 0)),
            pl.BlockSpec((1, br, DIM), lambda i: (1, i, 0)),
        ],
        out_specs=pl.BlockSpec((br, DIM), lambda i: (i, 0)),
        out_shape=jax.ShapeDtypeStruct((N_NODES, DIM), jnp.float32),
    )(p, p)


def kernel(x, node_keep_mask, source, target, edge_type, blocks):
    maskf = node_keep_mask.astype(jnp.float32).reshape(N_NODES, 1)
    source = source.astype(jnp.int32)
    target = target.astype(jnp.int32)
    edge_type = edge_type.astype(jnp.int32)

    gidx = jnp.concatenate([edge_type * ACC_ROWS + source,
                            edge_type * ACC_ROWS + target])
    sidx = jnp.concatenate([target, source])
    pad = jnp.arange(PAD, dtype=jnp.int32)
    gidx = jnp.concatenate([gidx, (pad * 131) % (N_REL * ACC_ROWS)])
    sidx = jnp.concatenate([sidx, N_NODES + (pad % (ACC_ROWS - N_NODES))])

    yflat = _transform(x, maskf, blocks)
    zeros = jnp.zeros((ACC_ROWS, DIM), jnp.float32)
    p = _sc_scatter(yflat, gidx, sidx, zeros)
    return _combine(p)


# speedup vs baseline: 1.1580x; 1.0039x over previous
import functools

import jax
import jax.numpy as jnp
from jax import lax
from jax.experimental import pallas as pl
from jax.experimental.pallas import tpu as pltpu
from jax.experimental.pallas import tpu_sc as plsc

N_NODES = 10000
DIM = 128
N_REL = 4
N_BLK = 4
BS = 32
N_EDGES = 160000

NW = 32
CHUNK = 128
E2 = 2 * N_EDGES
K = 4 * (-(-E2 // (NW * CHUNK * 4)))
E2P = NW * K * CHUNK
PAD = E2P - E2
ACC_ROWS = 10112
ROWS_PER_TILE = ACC_ROWS // 16


def _transform_body(x_ref, m_ref, blk_ref, y_ref):
    r = pl.program_id(0)
    xb = x_ref[...]
    z = jnp.zeros((BS, BS), jnp.float32)
    w = jnp.concatenate([
        jnp.concatenate([blk_ref[0, b] if j == b else z
                         for j in range(N_BLK)], axis=1)
        for b in range(N_BLK)], axis=0)
    yr = jnp.dot(xb, w, preferred_element_type=jnp.float32)
    y_ref[pl.ds(0, N_NODES), :] = jnp.where(r == N_REL, yr * m_ref[...], yr)


def _transform(x, maskf, blocks):
    return pl.pallas_call(
        _transform_body,
        grid=(N_REL + 1,),
        in_specs=[
            pl.BlockSpec((N_NODES, DIM), lambda r: (0, 0)),
            pl.BlockSpec((N_NODES, 1), lambda r: (0, 0)),
            pl.BlockSpec((1, N_BLK, BS, BS), lambda r: (r, 0, 0, 0)),
        ],
        out_specs=pl.BlockSpec((ACC_ROWS, DIM), lambda r: (r, 0)),
        out_shape=jax.ShapeDtypeStruct(((N_REL + 1) * ACC_ROWS, DIM),
                                       jnp.float32),
    )(x, maskf, blocks)


SOFF = N_REL * ACC_ROWS


def _sc_scatter(yflat, gidx, sidx, zeros):
    mesh = plsc.VectorSubcoreMesh(core_axis_name="c", subcore_axis_name="s")

    @functools.partial(
        pl.kernel,
        mesh=mesh,
        out_type=jax.ShapeDtypeStruct((2, ACC_ROWS, DIM), jnp.float32),
        scratch_types=[
            pltpu.VMEM((4, 2, CHUNK), jnp.int32),
            pltpu.VMEM((CHUNK, DIM), jnp.float32),
            pltpu.VMEM((CHUNK, DIM), jnp.float32),
            pltpu.VMEM_SHARED((ACC_ROWS, DIM), jnp.float32),
            pltpu.SemaphoreType.DMA,
            pltpu.SemaphoreType.DMA,
            pltpu.SemaphoreType.DMA,
            pltpu.SemaphoreType.DMA,
            pltpu.SemaphoreType.DMA,
            pltpu.SemaphoreType.DMA,
            pltpu.SemaphoreType.DMA,
            pltpu.SemaphoreType.DMA,
        ],
    )
    def body(yflat_hbm, gidx_hbm, sidx_hbm, zeros_hbm, out_hbm,
             ring, rows_a, rows_b, acc, g0, g1, s0, s1, i0, i1, i2, i3):
        cid = lax.axis_index("c")
        sid = lax.axis_index("s")
        wid = cid * 16 + sid
        base = wid * (K * CHUNK)
        rows = (rows_a, rows_b)
        gsem = (g0, g1)
        ssem = (s0, s1)
        isem = (i0, i1, i2, i3)

        def idx_start(row, slot):
            sl = pl.ds(base + row * CHUNK, CHUNK)
            pltpu.async_copy(gidx_hbm.at[sl], ring.at[slot, 0], isem[slot])
            pltpu.async_copy(sidx_hbm.at[sl], ring.at[slot, 1], isem[slot])

        def idx_wait(row, slot):
            sl = pl.ds(base + row * CHUNK, CHUNK)
            pltpu.make_async_copy(gidx_hbm.at[sl], ring.at[slot, 0],
                                  isem[slot]).wait()
            pltpu.make_async_copy(sidx_hbm.at[sl], ring.at[slot, 1],
                                  isem[slot]).wait()

        for u in range(4):
            idx_start(u, u)
        idx_wait(0, 0)
        pltpu.async_copy(yflat_hbm.at[ring.at[0, 0]], rows[0], gsem[0])

        sl_init = pl.ds(sid * ROWS_PER_TILE, ROWS_PER_TILE)

        @pl.when(cid == 0)
        def _():
            pltpu.sync_copy(
                yflat_hbm.at[pl.ds(SOFF + sid * ROWS_PER_TILE,
                                   ROWS_PER_TILE)],
                acc.at[sl_init])

        @pl.when(cid == 1)
        def _():
            pltpu.sync_copy(zeros_hbm, acc.at[sl_init])

        plsc.subcore_barrier()


        def quad(i, carry):
            for u in range(4):
                j = 4 * i + u
                sl, slp, sl1 = u, (u + 3) % 4, (u + 1) % 4
                bj, bj1 = rows[u % 2], rows[(u + 1) % 2]
                pltpu.make_async_copy(yflat_hbm.at[ring.at[sl, 0]],
                                      bj, gsem[u % 2]).wait()
                pltpu.async_copy(bj, acc.at[ring.at[sl, 1]], ssem[u % 2],
                                 add=True)

                @pl.when(j > 0)
                def _():
                    pltpu.make_async_copy(bj1, acc.at[ring.at[slp, 1]],
                                          ssem[(u + 1) % 2]).wait()

                @pl.when(jnp.logical_and(j > 0, j + 3 < K))
                def _():
                    idx_start(j + 3, slp)

                @pl.when(j + 1 < K)
                def _():
                    idx_wait(j + 1, sl1)
                    pltpu.async_copy(yflat_hbm.at[ring.at[sl1, 0]],
                                     bj1, gsem[(u + 1) % 2])
            return carry

        lax.fori_loop(0, K // 4, quad, 0)
        pltpu.make_async_copy(rows[(K - 1) % 2],
                              acc.at[ring.at[(K - 1) % 4, 1]],
                              ssem[(K - 1) % 2]).wait()
        plsc.subcore_barrier()
        pltpu.sync_copy(
            acc.at[pl.ds(sid * ROWS_PER_TILE, ROWS_PER_TILE)],
            out_hbm.at[cid].at[pl.ds(sid * ROWS_PER_TILE, ROWS_PER_TILE)])

    return body(yflat, gidx, sidx, zeros)


def _combine_body(p0_ref, p1_ref, o_ref):
    o_ref[...] = p0_ref[0] + p1_ref[0]


def _combine(p):
    br = 2000
    return pl.pallas_call(
        _combine_body,
        grid=(N_NODES // br,),
        in_specs=[
            pl.BlockSpec((1, br, DIM), lambda i: (0, i, 0)),
            pl.BlockSpec((1, br, DIM), lambda i: (1, i, 0)),
        ],
        out_specs=pl.BlockSpec((br, DIM), lambda i: (i, 0)),
        out_shape=jax.ShapeDtypeStruct((N_NODES, DIM), jnp.float32),
    )(p, p)


def kernel(x, node_keep_mask, source, target, edge_type, blocks):
    maskf = node_keep_mask.astype(jnp.float32).reshape(N_NODES, 1)
    source = source.astype(jnp.int32)
    target = target.astype(jnp.int32)
    edge_type = edge_type.astype(jnp.int32)

    gidx = jnp.concatenate([edge_type * ACC_ROWS + source,
                            edge_type * ACC_ROWS + target])
    sidx = jnp.concatenate([target, source])
    pad = jnp.arange(PAD, dtype=jnp.int32)
    gidx = jnp.concatenate([gidx, (pad * 131) % (N_REL * ACC_ROWS)])
    sidx = jnp.concatenate([sidx, N_NODES + (pad % (ACC_ROWS - N_NODES))])

    yflat = _transform(x, maskf, blocks)
    zeros = jnp.zeros((ROWS_PER_TILE, DIM), jnp.float32)
    p = _sc_scatter(yflat, gidx, sidx, zeros)
    return _combine(p)
